# baseline (speedup 1.0000x reference)
import functools

import jax
import jax.numpy as jnp
from jax import lax
from jax.experimental import pallas as pl
from jax.experimental.pallas import tpu as pltpu
from jax.experimental.pallas import tpu_sc as plsc

_NC = 2
_NS = 16
_NW = _NC * _NS

_CH = 128
_NFIELD = 26
_BATCH = 16384
_BW = _BATCH // _NW
_NCHF = _BW // _CH
_L = 16


@functools.partial(jax.jit, static_argnums=(2,))
def _sc_gather(embedding, idx, d):
  mesh = plsc.VectorSubcoreMesh(core_axis_name="c", subcore_axis_name="s")

  @functools.partial(
      pl.kernel,
      out_type=jax.ShapeDtypeStruct((_NFIELD, _BATCH, d), jnp.float32),
      mesh=mesh,
      scratch_types=[
          pltpu.VMEM((_BW, _NFIELD), jnp.int32),
          pltpu.VMEM((_NFIELD, _BW), jnp.int32),
          pltpu.VMEM((2, _BW, d), jnp.float32),
          pltpu.SemaphoreType.DMA,
          pltpu.SemaphoreType.DMA,
          pltpu.SemaphoreType.DMA,
          pltpu.SemaphoreType.DMA,
      ],
      compiler_params=pltpu.CompilerParams(use_tc_tiling_on_sc=False,
                                           needs_layout_passes=False),
  )
  def k(table_hbm, idx_hbm, out_hbm, idx_raw, idx_v, rows_v,
        sem0, sem1, wsem0, wsem1):
    wid = lax.axis_index("s") * _NC + lax.axis_index("c")
    base = wid * _BW
    pltpu.sync_copy(idx_hbm.at[pl.ds(base, _BW)], idx_raw)

    lane = lax.iota(jnp.int32, _L)
    for j0 in range(_BW // _L):
      rows = j0 * _L + lane

      def tbody(f, carry):
        cols = jnp.full((_L,), 0, jnp.int32) + f
        vals = plsc.load_gather(idx_raw, [rows, cols])
        idx_v[f, pl.ds(j0 * _L, _L)] = vals
        return carry

      lax.fori_loop(0, _NFIELD, tbody, 0)

    def fire_field(f, p, sem):
      for c in range(_NCHF):
        pltpu.async_copy(
            table_hbm.at[idx_v.at[f, pl.ds(c * _CH, _CH)]],
            rows_v.at[p, pl.ds(c * _CH, _CH)],
            sem,
        )

    def drain_field(p, sem):
      pltpu.make_async_copy(
          out_hbm.at[0, pl.ds(base, _BW)], rows_v.at[p], sem).wait()

    fire_field(0, 0, sem0)
    fire_field(1, 1, sem1)

    def body(i, carry):
      f = 2 * i
      drain_field(0, sem0)
      w0 = pltpu.async_copy(rows_v.at[0], out_hbm.at[f, pl.ds(base, _BW)],
                            wsem0)
      drain_field(1, sem1)
      w1 = pltpu.async_copy(rows_v.at[1], out_hbm.at[f + 1, pl.ds(base, _BW)],
                            wsem1)
      w0.wait()
      fire_field(f + 2, 0, sem0)
      w1.wait()
      fire_field(f + 3, 1, sem1)
      return carry

    lax.fori_loop(0, _NFIELD // 2 - 1, body, 0)

    drain_field(0, sem0)
    pltpu.async_copy(rows_v.at[0], out_hbm.at[_NFIELD - 2, pl.ds(base, _BW)],
                     wsem0)
    drain_field(1, sem1)
    pltpu.async_copy(rows_v.at[1], out_hbm.at[_NFIELD - 1, pl.ds(base, _BW)],
                     wsem1)
    pltpu.make_async_copy(rows_v.at[0], out_hbm.at[0, pl.ds(base, _BW)],
                          wsem0).wait()
    pltpu.make_async_copy(rows_v.at[1], out_hbm.at[0, pl.ds(base, _BW)],
                          wsem1).wait()

  return k(embedding, idx)


def kernel(inputs, embedding):
  b, f = inputs.shape
  d = embedding.shape[1]
  idx = inputs if inputs.dtype == jnp.int32 else inputs.astype(jnp.int32)
  out = _sc_gather(embedding, idx, d)
  return out.transpose(1, 0, 2)
